# baseline (speedup 1.0000x reference)
import jax
import jax.numpy as jnp
from jax import lax
from jax.experimental import pallas as pl
from jax.experimental.pallas import tpu as pltpu
from jax.experimental.pallas import tpu_sc as plsc

N = 10000
E = 640000
H = 64
G = 128

NC = 2
NS = 16
NW = NC * NS
EB = 40
E_PAD = 655360
EPW = E_PAD // NW
NBLK = EPW // EB
GB = EB
N_PAD = 10240
ROWS_PER_TILE = N_PAD // NS
PAD_IDX = 10016


def _embed_body(x_ref, we_ref, be_ref, wd_ref, bd_ref, ws_ref,
                h_ref, td_ref, ts_ref):
    h = jax.nn.relu(
        jnp.dot(x_ref[...], we_ref[...], preferred_element_type=jnp.float32)
        + be_ref[...])
    h_ref[...] = h
    td_ref[...] = jnp.dot(h, wd_ref[...], preferred_element_type=jnp.float32) + bd_ref[...]
    ts_ref[...] = jnp.dot(h, ws_ref[...], preferred_element_type=jnp.float32)


def _embed(x, W_emb, b_emb, Wd, bd, Ws):
    BLK = 2048
    return pl.pallas_call(
        _embed_body,
        grid=(N_PAD // BLK,),
        in_specs=[
            pl.BlockSpec((BLK, 128), lambda i: (i, 0)),
            pl.BlockSpec((128, H), lambda i: (0, 0)),
            pl.BlockSpec((1, H), lambda i: (0, 0)),
            pl.BlockSpec((H, 128), lambda i: (0, 0)),
            pl.BlockSpec((1, 128), lambda i: (0, 0)),
            pl.BlockSpec((H, 128), lambda i: (0, 0)),
        ],
        out_specs=[
            pl.BlockSpec((BLK, H), lambda i: (i, 0)),
            pl.BlockSpec((BLK, 128), lambda i: (i, 0)),
            pl.BlockSpec((BLK, 128), lambda i: (i, 0)),
        ],
        out_shape=[
            jax.ShapeDtypeStruct((N_PAD, H), jnp.float32),
            jax.ShapeDtypeStruct((N_PAD, 128), jnp.float32),
            jax.ShapeDtypeStruct((N_PAD, 128), jnp.float32),
        ],
    )(x, W_emb, b_emb, Wd, bd, Ws)


def _edgeproj_body(ea_ref, w1_ref, w2_ref, o1_ref, o2_ref):
    ea = ea_ref[...]
    o1_ref[...] = jnp.dot(ea, w1_ref[...], preferred_element_type=jnp.float32)
    o2_ref[...] = jnp.dot(ea, w2_ref[...], preferred_element_type=jnp.float32)


def _edgeproj(edge_attr, We1, We2):
    BLK = 8192
    return pl.pallas_call(
        _edgeproj_body,
        grid=(E_PAD // BLK,),
        in_specs=[
            pl.BlockSpec((BLK, 16), lambda i: (i, 0)),
            pl.BlockSpec((16, 128), lambda i: (0, 0)),
            pl.BlockSpec((16, 128), lambda i: (0, 0)),
        ],
        out_specs=[
            pl.BlockSpec((BLK, 128), lambda i: (i, 0)),
            pl.BlockSpec((BLK, 128), lambda i: (i, 0)),
        ],
        out_shape=[
            jax.ShapeDtypeStruct((E_PAD, 128), jnp.float32),
            jax.ShapeDtypeStruct((E_PAD, 128), jnp.float32),
        ],
    )(edge_attr, We1, We2)


def _mid_body(h_ref, agg_ref, wd_ref, bd_ref, ws_ref, h1_ref, td_ref, ts_ref):
    h1 = jax.nn.relu(h_ref[...] + agg_ref[0, :, :H] + agg_ref[1, :, :H])
    h1_ref[...] = h1
    td_ref[...] = jnp.dot(h1, wd_ref[...], preferred_element_type=jnp.float32) + bd_ref[...]
    ts_ref[...] = jnp.dot(h1, ws_ref[...], preferred_element_type=jnp.float32)


def _mid(h, agg, Wd, bd, Ws):
    BLK = 2048
    return pl.pallas_call(
        _mid_body,
        grid=(N_PAD // BLK,),
        in_specs=[
            pl.BlockSpec((BLK, H), lambda i: (i, 0)),
            pl.BlockSpec((NC, BLK, 128), lambda i: (0, i, 0)),
            pl.BlockSpec((H, 128), lambda i: (0, 0)),
            pl.BlockSpec((1, 128), lambda i: (0, 0)),
            pl.BlockSpec((H, 128), lambda i: (0, 0)),
        ],
        out_specs=[
            pl.BlockSpec((BLK, H), lambda i: (i, 0)),
            pl.BlockSpec((BLK, 128), lambda i: (i, 0)),
            pl.BlockSpec((BLK, 128), lambda i: (i, 0)),
        ],
        out_shape=[
            jax.ShapeDtypeStruct((N_PAD, H), jnp.float32),
            jax.ShapeDtypeStruct((N_PAD, 128), jnp.float32),
            jax.ShapeDtypeStruct((N_PAD, 128), jnp.float32),
        ],
    )(h, agg, Wd, bd, Ws)


def _pool_body(h_ref, agg_ref, b_ref, w1_ref, b1_ref, w2_ref, b2_ref,
               out_ref, sums_ref, cnt_ref):
    i = pl.program_id(0)
    nblk = pl.num_programs(0)

    @pl.when(i == 0)
    def _():
        sums_ref[...] = jnp.zeros_like(sums_ref)
        cnt_ref[...] = jnp.zeros_like(cnt_ref)

    h2 = jax.nn.relu(h_ref[...] + agg_ref[0, :, :H] + agg_ref[1, :, :H])
    b = b_ref[0, 0, :]
    gids = lax.broadcasted_iota(jnp.int32, (h2.shape[0], G), 1)
    oh = (b[:, None] == gids).astype(jnp.float32)
    dn = (((0,), (0,)), ((), ()))
    sums_ref[...] += lax.dot_general(oh, h2, dn,
                                     preferred_element_type=jnp.float32)
    ones = jnp.ones((h2.shape[0], G), dtype=jnp.float32)
    cnt_ref[...] += lax.dot_general(oh, ones, dn,
                                    preferred_element_type=jnp.float32)

    @pl.when(i == nblk - 1)
    def _():
        pooled = sums_ref[...] / jnp.maximum(cnt_ref[:, :H], 1.0)
        hid = jax.nn.relu(
            jnp.dot(pooled, w1_ref[...], preferred_element_type=jnp.float32)
            + b1_ref[...])
        out_ref[...] = jnp.dot(hid, w2_ref[...],
                               preferred_element_type=jnp.float32) + b2_ref[...]


def _pool(h, agg, batch_r, W1, b1, W2, b2):
    BLK = 2048
    return pl.pallas_call(
        _pool_body,
        grid=(N_PAD // BLK,),
        in_specs=[
            pl.BlockSpec((BLK, H), lambda i: (i, 0)),
            pl.BlockSpec((NC, BLK, 128), lambda i: (0, i, 0)),
            pl.BlockSpec((1, 1, BLK), lambda i: (i, 0, 0)),
            pl.BlockSpec((H, 32), lambda i: (0, 0)),
            pl.BlockSpec((1, 32), lambda i: (0, 0)),
            pl.BlockSpec((32, 2), lambda i: (0, 0)),
            pl.BlockSpec((1, 2), lambda i: (0, 0)),
        ],
        out_specs=pl.BlockSpec((G, 2), lambda i: (0, 0)),
        out_shape=jax.ShapeDtypeStruct((G, 2), jnp.float32),
        scratch_shapes=[
            pltpu.VMEM((G, H), jnp.float32),
            pltpu.VMEM((G, G), jnp.float32),
        ],
    )(h, agg, batch_r, W1, b1, W2, b2)


def _msg_sub(dbuf, sbuf, ebuf, mbuf, q):
    def body(j, _):
        for c in range(4):
            sl = pl.ds(16 * c, 16)
            sh = pl.ds(64 + 16 * c, 16)
            u = dbuf[j, sl] + sbuf[j, sl] + ebuf[j, sl]
            v = dbuf[j, sh] + sbuf[j, sh] + ebuf[j, sh]
            f = 1.0 / (1.0 + jnp.exp(-u))
            e = jnp.exp(-jnp.abs(v))
            t = e / (2.0 + e)
            t2 = t * t
            L = t * (2.0 + t2 * (2.0 / 3.0 + t2 * (2.0 / 5.0 + t2 * (
                2.0 / 7.0 + t2 * (2.0 / 9.0 + t2 * (2.0 / 11.0))))))
            mbuf[q * GB + j, sl] = f * (jnp.maximum(v, 0.0) + L)
        return 0
    lax.fori_loop(0, GB, body, 0)


def _conv_sc_body(td_hbm, ts_hbm, ep_hbm, dst_hbm, src_hbm, zer_hbm, out_hbm,
                  didx0, sidx0, didx1, sidx1,
                  dbuf0, sbuf0, ebuf0, dbuf1, sbuf1, ebuf1,
                  mbuf, agg_sh,
                  semi0, semi1, semd0, sems0, seme0, semd1, sems1, seme1):
    cid = lax.axis_index("c")
    sid = lax.axis_index("s")
    wid = cid * NS + sid
    base0 = wid * EPW
    didx = (didx0, didx1)
    sidx = (sidx0, sidx1)
    dbuf = (dbuf0, dbuf1)
    sbuf = (sbuf0, sbuf1)
    ebuf = (ebuf0, ebuf1)
    semi = (semi0, semi1)
    semd = (semd0, semd1)
    sems = (sems0, sems1)
    seme = (seme0, seme1)

    def load_idx(base, i):
        base = jnp.minimum(base, E_PAD - EB)
        pltpu.async_copy(dst_hbm.at[pl.ds(base, EB)], didx[i], semi[i])
        pltpu.async_copy(src_hbm.at[pl.ds(base, EB)], sidx[i], semi[i])

    def wait_idx(i):
        pltpu.make_async_copy(dst_hbm.at[pl.ds(0, EB)], didx[i], semi[i]).wait()
        pltpu.make_async_copy(src_hbm.at[pl.ds(0, EB)], sidx[i], semi[i]).wait()

    def gath(base, g):
        base = jnp.minimum(base, E_PAD - EB)
        pltpu.async_copy(td_hbm.at[didx[g]], dbuf[g], semd[g])
        pltpu.async_copy(ts_hbm.at[sidx[g]], sbuf[g], sems[g])
        pltpu.async_copy(ep_hbm.at[pl.ds(base, EB), :], ebuf[g], seme[g])

    def wait_gath(g):
        pltpu.make_async_copy(td_hbm.at[didx[g]], dbuf[g], semd[g]).wait()
        pltpu.make_async_copy(ts_hbm.at[sidx[g]], sbuf[g], sems[g]).wait()
        pltpu.make_async_copy(ep_hbm.at[pl.ds(0, EB), :], ebuf[g], seme[g]).wait()

    def consume(g):
        _msg_sub(dbuf[g], sbuf[g], ebuf[g], mbuf, 0)
        pltpu.sync_copy(mbuf, agg_sh.at[didx[g]], add=True)

    def zmb(j, _):
        for c in range(4, 8):
            mbuf[j, pl.ds(16 * c, 16)] = jnp.zeros((16,), jnp.float32)
        return 0
    lax.fori_loop(0, EB, zmb, 0)

    row0 = sid * ROWS_PER_TILE
    pltpu.sync_copy(zer_hbm.at[pl.ds(row0, ROWS_PER_TILE)],
                    agg_sh.at[pl.ds(row0, ROWS_PER_TILE)])
    plsc.subcore_barrier()

    load_idx(base0, 0)
    load_idx(base0 + EB, 1)
    wait_idx(0)
    gath(base0, 0)

    def pair(kk, _):
        b0 = base0 + kk * (2 * EB)
        wait_idx(1)
        gath(b0 + EB, 1)
        wait_gath(0)
        consume(0)
        load_idx(b0 + 2 * EB, 0)
        wait_idx(0)
        gath(b0 + 2 * EB, 0)
        wait_gath(1)
        consume(1)
        load_idx(b0 + 3 * EB, 1)
        return 0

    lax.fori_loop(0, NBLK // 2, pair, 0)
    wait_gath(0)
    wait_idx(1)

    plsc.subcore_barrier()
    pltpu.sync_copy(agg_sh.at[pl.ds(row0, ROWS_PER_TILE)],
                    out_hbm.at[cid, pl.ds(row0, ROWS_PER_TILE)])


def _conv_sc(Td, Ts, EP, dstp, srcp, zeros_nh):
    mesh = plsc.VectorSubcoreMesh(core_axis_name="c", subcore_axis_name="s")
    f = pl.kernel(
        _conv_sc_body,
        out_type=jax.ShapeDtypeStruct((NC, N_PAD, 128), jnp.float32),
        mesh=mesh,
        scratch_types=(
            [pltpu.VMEM((EB,), jnp.int32)] * 4
            + [pltpu.VMEM((EB, 128), jnp.float32)] * 6
            + [pltpu.VMEM((EB, 128), jnp.float32),
               pltpu.VMEM_SHARED((N_PAD, 128), jnp.float32)]
            + [pltpu.SemaphoreType.DMA] * 8
        ),
    )
    return f(Td, Ts, EP, dstp, srcp, zeros_nh)


def kernel(x, edge_index, edge_attr, batch, W_emb, b_emb, Wf1, bf1, Ws1, bs1,
           Wf2, bf2, Ws2, bs2, W1, b1, W2, b2):
    src = edge_index[0]
    dst = edge_index[1]
    epad = E_PAD - E
    npad = N_PAD - N
    dstp = jnp.concatenate([dst, jnp.full((epad,), PAD_IDX, jnp.int32)])
    srcp = jnp.concatenate([src, jnp.full((epad,), PAD_IDX, jnp.int32)])
    eap = jnp.concatenate([edge_attr, jnp.zeros((epad, 16), jnp.float32)])
    xp = jnp.concatenate([x, jnp.zeros((npad, 128), jnp.float32)])
    batchp = jnp.concatenate([batch, jnp.full((npad,), G, jnp.int32)])

    Wd1 = jnp.concatenate([Wf1[:H], Ws1[:H]], axis=1)
    bd1 = jnp.concatenate([bf1, bs1]).reshape(1, 128)
    Wsr1 = jnp.concatenate([Wf1[H:2 * H], Ws1[H:2 * H]], axis=1)
    We1 = jnp.concatenate([Wf1[2 * H:], Ws1[2 * H:]], axis=1)
    Wd2 = jnp.concatenate([Wf2[:H], Ws2[:H]], axis=1)
    bd2 = jnp.concatenate([bf2, bs2]).reshape(1, 128)
    Wsr2 = jnp.concatenate([Wf2[H:2 * H], Ws2[H:2 * H]], axis=1)
    We2 = jnp.concatenate([Wf2[2 * H:], Ws2[2 * H:]], axis=1)

    zeros_nh = jnp.zeros((N_PAD, 128), jnp.float32)

    h0, Td1, Ts1 = _embed(xp, W_emb, b_emb.reshape(1, H), Wd1, bd1, Wsr1)
    EP1, EP2 = _edgeproj(eap, We1, We2)
    agg1 = _conv_sc(Td1, Ts1, EP1, dstp, srcp, zeros_nh)
    h1, Td2, Ts2 = _mid(h0, agg1, Wd2, bd2, Wsr2)
    agg2 = _conv_sc(Td2, Ts2, EP2, dstp, srcp, zeros_nh)
    out = _pool(h1, agg2, batchp.reshape(N_PAD // 2048, 1, 2048),
                W1, b1.reshape(1, 32), W2, b2.reshape(1, 2))
    return out


# speedup vs baseline: 1.4181x; 1.4181x over previous
import jax
import jax.numpy as jnp
from jax import lax
from jax.experimental import pallas as pl
from jax.experimental.pallas import tpu as pltpu
from jax.experimental.pallas import tpu_sc as plsc

N = 10000
E = 640000
H = 64
G = 128

NC = 2
NS = 16
NW = NC * NS
EB = 40
E_PAD = E
EPW = E_PAD // NW
NBLK = EPW // EB
GB = EB
N_PAD = 10240
ROWS_PER_TILE = N_PAD // NS
PAD_IDX = 10016


def _embed_body(x_ref, we_ref, be_ref, wd_ref, bd_ref, ws_ref,
                h_ref, td_ref, ts_ref):
    h = jax.nn.relu(
        jnp.dot(x_ref[...], we_ref[...], preferred_element_type=jnp.float32)
        + be_ref[...])
    h_ref[...] = h
    td_ref[...] = jnp.dot(h, wd_ref[...], preferred_element_type=jnp.float32) + bd_ref[...]
    ts_ref[...] = jnp.dot(h, ws_ref[...], preferred_element_type=jnp.float32)


def _embed(x, W_emb, b_emb, Wd, bd, Ws):
    BLK = 2048
    return pl.pallas_call(
        _embed_body,
        grid=(N_PAD // BLK,),
        in_specs=[
            pl.BlockSpec((BLK, 128), lambda i: (i, 0)),
            pl.BlockSpec((128, H), lambda i: (0, 0)),
            pl.BlockSpec((1, H), lambda i: (0, 0)),
            pl.BlockSpec((H, 128), lambda i: (0, 0)),
            pl.BlockSpec((1, 128), lambda i: (0, 0)),
            pl.BlockSpec((H, 128), lambda i: (0, 0)),
        ],
        out_specs=[
            pl.BlockSpec((BLK, H), lambda i: (i, 0)),
            pl.BlockSpec((BLK, 128), lambda i: (i, 0)),
            pl.BlockSpec((BLK, 128), lambda i: (i, 0)),
        ],
        out_shape=[
            jax.ShapeDtypeStruct((N_PAD, H), jnp.float32),
            jax.ShapeDtypeStruct((N_PAD, 128), jnp.float32),
            jax.ShapeDtypeStruct((N_PAD, 128), jnp.float32),
        ],
    )(x, W_emb, b_emb, Wd, bd, Ws)


def _edgeproj_body(ea_ref, w1_ref, w2_ref, o1_ref, o2_ref):
    ea = ea_ref[...]
    o1_ref[...] = jnp.dot(ea, w1_ref[...], preferred_element_type=jnp.float32)
    o2_ref[...] = jnp.dot(ea, w2_ref[...], preferred_element_type=jnp.float32)


def _edgeproj(edge_attr, We1, We2):
    BLK = 8192
    return pl.pallas_call(
        _edgeproj_body,
        grid=(E_PAD // BLK,),
        in_specs=[
            pl.BlockSpec((BLK, 16), lambda i: (i, 0)),
            pl.BlockSpec((16, 128), lambda i: (0, 0)),
            pl.BlockSpec((16, 128), lambda i: (0, 0)),
        ],
        out_specs=[
            pl.BlockSpec((BLK, 128), lambda i: (i, 0)),
            pl.BlockSpec((BLK, 128), lambda i: (i, 0)),
        ],
        out_shape=[
            jax.ShapeDtypeStruct((E_PAD, 128), jnp.float32),
            jax.ShapeDtypeStruct((E_PAD, 128), jnp.float32),
        ],
    )(edge_attr, We1, We2)


def _mid_body(h_ref, agg_ref, wd_ref, bd_ref, ws_ref, h1_ref, td_ref, ts_ref):
    h1 = jax.nn.relu(h_ref[...] + agg_ref[0, :, :H] + agg_ref[1, :, :H])
    h1_ref[...] = h1
    td_ref[...] = jnp.dot(h1, wd_ref[...], preferred_element_type=jnp.float32) + bd_ref[...]
    ts_ref[...] = jnp.dot(h1, ws_ref[...], preferred_element_type=jnp.float32)


def _mid(h, agg, Wd, bd, Ws):
    BLK = 2048
    return pl.pallas_call(
        _mid_body,
        grid=(N_PAD // BLK,),
        in_specs=[
            pl.BlockSpec((BLK, H), lambda i: (i, 0)),
            pl.BlockSpec((NC, BLK, 128), lambda i: (0, i, 0)),
            pl.BlockSpec((H, 128), lambda i: (0, 0)),
            pl.BlockSpec((1, 128), lambda i: (0, 0)),
            pl.BlockSpec((H, 128), lambda i: (0, 0)),
        ],
        out_specs=[
            pl.BlockSpec((BLK, H), lambda i: (i, 0)),
            pl.BlockSpec((BLK, 128), lambda i: (i, 0)),
            pl.BlockSpec((BLK, 128), lambda i: (i, 0)),
        ],
        out_shape=[
            jax.ShapeDtypeStruct((N_PAD, H), jnp.float32),
            jax.ShapeDtypeStruct((N_PAD, 128), jnp.float32),
            jax.ShapeDtypeStruct((N_PAD, 128), jnp.float32),
        ],
    )(h, agg, Wd, bd, Ws)


def _pool_body(h_ref, agg_ref, b_ref, w1_ref, b1_ref, w2_ref, b2_ref,
               out_ref, sums_ref, cnt_ref):
    i = pl.program_id(0)
    nblk = pl.num_programs(0)

    @pl.when(i == 0)
    def _():
        sums_ref[...] = jnp.zeros_like(sums_ref)
        cnt_ref[...] = jnp.zeros_like(cnt_ref)

    h2 = jax.nn.relu(h_ref[...] + agg_ref[0, :, :H] + agg_ref[1, :, :H])
    b = b_ref[0, 0, :]
    gids = lax.broadcasted_iota(jnp.int32, (h2.shape[0], G), 1)
    oh = (b[:, None] == gids).astype(jnp.float32)
    dn = (((0,), (0,)), ((), ()))
    sums_ref[...] += lax.dot_general(oh, h2, dn,
                                     preferred_element_type=jnp.float32)
    ones = jnp.ones((h2.shape[0], G), dtype=jnp.float32)
    cnt_ref[...] += lax.dot_general(oh, ones, dn,
                                    preferred_element_type=jnp.float32)

    @pl.when(i == nblk - 1)
    def _():
        pooled = sums_ref[...] / jnp.maximum(cnt_ref[:, :H], 1.0)
        hid = jax.nn.relu(
            jnp.dot(pooled, w1_ref[...], preferred_element_type=jnp.float32)
            + b1_ref[...])
        out_ref[...] = jnp.dot(hid, w2_ref[...],
                               preferred_element_type=jnp.float32) + b2_ref[...]


def _pool(h, agg, batch_r, W1, b1, W2, b2):
    BLK = 2048
    return pl.pallas_call(
        _pool_body,
        grid=(N_PAD // BLK,),
        in_specs=[
            pl.BlockSpec((BLK, H), lambda i: (i, 0)),
            pl.BlockSpec((NC, BLK, 128), lambda i: (0, i, 0)),
            pl.BlockSpec((1, 1, BLK), lambda i: (i, 0, 0)),
            pl.BlockSpec((H, 32), lambda i: (0, 0)),
            pl.BlockSpec((1, 32), lambda i: (0, 0)),
            pl.BlockSpec((32, 2), lambda i: (0, 0)),
            pl.BlockSpec((1, 2), lambda i: (0, 0)),
        ],
        out_specs=pl.BlockSpec((G, 2), lambda i: (0, 0)),
        out_shape=jax.ShapeDtypeStruct((G, 2), jnp.float32),
        scratch_shapes=[
            pltpu.VMEM((G, H), jnp.float32),
            pltpu.VMEM((G, G), jnp.float32),
        ],
    )(h, agg, batch_r, W1, b1, W2, b2)


def _msg_sub(dbuf, sbuf, ebuf, mbuf, q):
    def body(j, _):
        for c in range(4):
            sl = pl.ds(16 * c, 16)
            sh = pl.ds(64 + 16 * c, 16)
            u = dbuf[j, sl] + sbuf[j, sl] + ebuf[j, sl]
            v = dbuf[j, sh] + sbuf[j, sh] + ebuf[j, sh]
            f = 1.0 / (1.0 + jnp.exp(-u))
            e = jnp.exp(-jnp.abs(v))
            t = e / (2.0 + e)
            t2 = t * t
            L = t * (2.0 + t2 * (2.0 / 3.0 + t2 * (2.0 / 5.0 + t2 * (
                2.0 / 7.0 + t2 * (2.0 / 9.0 + t2 * (2.0 / 11.0))))))
            mbuf[q * GB + j, sl] = f * (jnp.maximum(v, 0.0) + L)
        return 0
    lax.fori_loop(0, GB, body, 0)


def _conv_sc_body(td_hbm, ts_hbm, ep_hbm, dst_hbm, src_hbm, zer_hbm, out_hbm,
                  didx0, sidx0, didx1, sidx1,
                  dbuf0, sbuf0, ebuf0, dbuf1, sbuf1, ebuf1,
                  mbuf, agg_sh,
                  semi0, semi1, semd0, sems0, seme0, semd1, sems1, seme1):
    cid = lax.axis_index("c")
    sid = lax.axis_index("s")
    wid = cid * NS + sid
    base0 = wid * EPW
    didx = (didx0, didx1)
    sidx = (sidx0, sidx1)
    dbuf = (dbuf0, dbuf1)
    sbuf = (sbuf0, sbuf1)
    ebuf = (ebuf0, ebuf1)
    semi = (semi0, semi1)
    semd = (semd0, semd1)
    sems = (sems0, sems1)
    seme = (seme0, seme1)

    def load_idx(base, i):
        base = jnp.minimum(base, E_PAD - EB)
        pltpu.async_copy(dst_hbm.at[pl.ds(base, EB)], didx[i], semi[i])
        pltpu.async_copy(src_hbm.at[pl.ds(base, EB)], sidx[i], semi[i])

    def wait_idx(i):
        pltpu.make_async_copy(dst_hbm.at[pl.ds(0, EB)], didx[i], semi[i]).wait()
        pltpu.make_async_copy(src_hbm.at[pl.ds(0, EB)], sidx[i], semi[i]).wait()

    def gath(base, g):
        base = jnp.minimum(base, E_PAD - EB)
        pltpu.async_copy(td_hbm.at[didx[g]], dbuf[g], semd[g])
        pltpu.async_copy(ts_hbm.at[sidx[g]], sbuf[g], sems[g])
        pltpu.async_copy(ep_hbm.at[pl.ds(base, EB), :], ebuf[g], seme[g])

    def wait_gath(g):
        pltpu.make_async_copy(td_hbm.at[didx[g]], dbuf[g], semd[g]).wait()
        pltpu.make_async_copy(ts_hbm.at[sidx[g]], sbuf[g], sems[g]).wait()
        pltpu.make_async_copy(ep_hbm.at[pl.ds(0, EB), :], ebuf[g], seme[g]).wait()

    def consume(g):
        _msg_sub(dbuf[g], sbuf[g], ebuf[g], mbuf, 0)
        pltpu.sync_copy(mbuf, agg_sh.at[didx[g]], add=True)

    def zmb(j, _):
        for c in range(4, 8):
            mbuf[j, pl.ds(16 * c, 16)] = jnp.zeros((16,), jnp.float32)
        return 0
    lax.fori_loop(0, EB, zmb, 0)

    row0 = sid * ROWS_PER_TILE
    pltpu.sync_copy(zer_hbm.at[pl.ds(row0, ROWS_PER_TILE)],
                    agg_sh.at[pl.ds(row0, ROWS_PER_TILE)])
    plsc.subcore_barrier()

    load_idx(base0, 0)
    load_idx(base0 + EB, 1)
    wait_idx(0)
    gath(base0, 0)

    def pair(kk, _):
        b0 = base0 + kk * (2 * EB)
        wait_idx(1)
        gath(b0 + EB, 1)
        wait_gath(0)
        consume(0)
        load_idx(b0 + 2 * EB, 0)
        wait_idx(0)
        gath(b0 + 2 * EB, 0)
        wait_gath(1)
        consume(1)
        load_idx(b0 + 3 * EB, 1)
        return 0

    lax.fori_loop(0, NBLK // 2, pair, 0)
    wait_gath(0)
    wait_idx(1)

    plsc.subcore_barrier()
    pltpu.sync_copy(agg_sh.at[pl.ds(row0, ROWS_PER_TILE)],
                    out_hbm.at[cid, pl.ds(row0, ROWS_PER_TILE)])


def _conv_sc(Td, Ts, EP, dstp, srcp, zeros_nh):
    mesh = plsc.VectorSubcoreMesh(core_axis_name="c", subcore_axis_name="s")
    f = pl.kernel(
        _conv_sc_body,
        out_type=jax.ShapeDtypeStruct((NC, N_PAD, 128), jnp.float32),
        mesh=mesh,
        scratch_types=(
            [pltpu.VMEM((EB,), jnp.int32)] * 4
            + [pltpu.VMEM((EB, 128), jnp.float32)] * 6
            + [pltpu.VMEM((EB, 128), jnp.float32),
               pltpu.VMEM_SHARED((N_PAD, 128), jnp.float32)]
            + [pltpu.SemaphoreType.DMA] * 8
        ),
    )
    return f(Td, Ts, EP, dstp, srcp, zeros_nh)


def kernel(x, edge_index, edge_attr, batch, W_emb, b_emb, Wf1, bf1, Ws1, bs1,
           Wf2, bf2, Ws2, bs2, W1, b1, W2, b2):
    src = edge_index[0]
    dst = edge_index[1]
    epad = E_PAD - E
    npad = N_PAD - N
    dstp = jnp.concatenate([dst, jnp.full((epad,), PAD_IDX, jnp.int32)])
    srcp = jnp.concatenate([src, jnp.full((epad,), PAD_IDX, jnp.int32)])
    eap = jnp.concatenate([edge_attr, jnp.zeros((epad, 16), jnp.float32)])
    xp = jnp.concatenate([x, jnp.zeros((npad, 128), jnp.float32)])
    batchp = jnp.concatenate([batch, jnp.full((npad,), G, jnp.int32)])

    Wd1 = jnp.concatenate([Wf1[:H], Ws1[:H]], axis=1)
    bd1 = jnp.concatenate([bf1, bs1]).reshape(1, 128)
    Wsr1 = jnp.concatenate([Wf1[H:2 * H], Ws1[H:2 * H]], axis=1)
    We1 = jnp.concatenate([Wf1[2 * H:], Ws1[2 * H:]], axis=1)
    Wd2 = jnp.concatenate([Wf2[:H], Ws2[:H]], axis=1)
    bd2 = jnp.concatenate([bf2, bs2]).reshape(1, 128)
    Wsr2 = jnp.concatenate([Wf2[H:2 * H], Ws2[H:2 * H]], axis=1)
    We2 = jnp.concatenate([Wf2[2 * H:], Ws2[2 * H:]], axis=1)

    zeros_nh = jnp.zeros((N_PAD, 128), jnp.float32)

    h0, Td1, Ts1 = _embed(xp, W_emb, b_emb.reshape(1, H), Wd1, bd1, Wsr1)
    EP1, EP2 = _edgeproj(eap, We1, We2)
    agg1 = _conv_sc(Td1, Ts1, EP1, dstp, srcp, zeros_nh)
    h1, Td2, Ts2 = _mid(h0, agg1, Wd2, bd2, Wsr2)
    agg2 = _conv_sc(Td2, Ts2, EP2, dstp, srcp, zeros_nh)
    out = _pool(h1, agg2, batchp.reshape(N_PAD // 2048, 1, 2048),
                W1, b1.reshape(1, 32), W2, b2.reshape(1, 2))
    return out
